# baseline (speedup 1.0000x reference)
import jax
import jax.numpy as jnp
from jax.experimental import pallas as pl
from jax.experimental.pallas import tpu as pltpu

H = 160
W = 128
N = H * W
KTOP = 2000
BUF = 2048
OUTN = 500
OUTP = 512
THR2 = 1.0
MIN_SCORE = 0.5
ONE_BITS = 0x3F800000

_f32 = jnp.float32


def _fiota(shape, dim):
    return jax.lax.broadcasted_iota(jnp.int32, shape, dim).astype(_f32)


def _dotT(a, b, exact=False):
    prec = jax.lax.Precision.HIGHEST if exact else jax.lax.Precision.DEFAULT
    return jax.lax.dot_general(
        a, b, (((1,), (1,)), ((), ())), precision=prec,
        preferred_element_type=_f32)


def _tril_strict(n):
    r = _fiota((n, n), 0)
    c = _fiota((n, n), 1)
    return (c < r).astype(_f32)


def _nms_body(scores_ref, reg_y_ref, reg_x_ref,
              out_s_ref, out_y_ref, out_x_ref,
              sel_s, pos_s, y_s, x_s, S_s, keep_s, acc_s):
    scores = scores_ref[0]
    ri = _fiota((H, W), 0)
    ci = _fiota((H, W), 1)
    y_s[...] = ri + 0.5 + reg_y_ref[0]
    x_s[...] = ci + 0.5 + reg_x_ref[0]

    s_bits = jax.lax.bitcast_convert_type(scores, jnp.int32)

    def bis(_, lohi):
        lo, hi = lohi
        m = lo + (hi - lo) // 2
        cnt = jnp.sum((s_bits >= m).astype(jnp.int32))
        big = cnt >= KTOP
        return (jnp.where(big, m, lo), jnp.where(big, hi, m))

    lo, _ = jax.lax.fori_loop(
        0, 31, bis, (jnp.int32(0), jnp.int32(ONE_BITS)))
    mask_gt = (s_bits > lo).astype(_f32)
    mask_eq = (s_bits == lo).astype(_f32)
    need_eq = _f32(KTOP) - jnp.sum(mask_gt)

    TT_W = _tril_strict(W)
    TL_H = _tril_strict(H)

    def excl2d(m2d):
        e = _dotT(m2d, TT_W)
        t = jnp.sum(m2d, axis=1, keepdims=True)
        roff = jax.lax.dot_general(
            TL_H, t, (((1,), (0,)), ((), ())), preferred_element_type=_f32)
        return roff + e

    eq_pos = excl2d(mask_eq)
    mask_sel = mask_gt + mask_eq * (eq_pos < need_eq).astype(_f32)
    sel_s[...] = mask_sel
    pos_s[...] = _dotT(mask_sel, TT_W)

    iq256 = _fiota((2 * W, 1), 0)
    acc_s[...] = jnp.zeros((3, BUF // W + 1, W), _f32)

    def crow(r, roff):
        s_r = scores_ref[0, pl.ds(r, 1), :]
        y_r = y_s[pl.ds(r, 1), :]
        x_r = x_s[pl.ds(r, 1), :]
        sel_r = sel_s[pl.ds(r, 1), :]
        e_r = pos_s[pl.ds(r, 1), :]
        hi = (roff.astype(jnp.int32)) // W
        lo = roff - (hi * W).astype(_f32)
        v_r = jnp.concatenate([s_r, y_r, x_r], axis=0)
        o_r = (iq256 == (e_r + lo)).astype(_f32) * sel_r
        part = _dotT(v_r, o_r, exact=True)
        p0 = part[:, 0:W].reshape(3, 1, W)
        p1 = part[:, W:2 * W].reshape(3, 1, W)
        acc_s[:, pl.ds(hi, 1), :] += p0
        acc_s[:, pl.ds(hi + 1, 1), :] += p1
        return roff + jnp.sum(sel_r)

    jax.lax.fori_loop(0, H, crow, _f32(0.0))
    accv = acc_s[...]
    flat = jnp.concatenate(
        [accv[:, g, :] for g in range(BUF // W)], axis=1)

    iq_col = _fiota((BUF, 1), 0)
    slot = _fiota((1, BUF), 1)
    real = slot < _f32(KTOP)
    sc = jnp.where(real, flat[0:1, :], -1.0)
    yc = jnp.where(real, flat[1:2, :], 1e6 + slot)
    xc = jnp.where(real, flat[2:3, :], 1e6)

    I_BUF = (_fiota((BUF, BUF), 0) ==
             _fiota((BUF, BUF), 1)).astype(_f32)
    vc = jnp.concatenate([sc, yc, xc], axis=0)
    cols = _dotT(I_BUF, vc, exact=True)
    s_col = cols[:, 0:1]
    y_col = cols[:, 1:2]
    x_col = cols[:, 2:3]
    g2 = (s_col > sc) | ((s_col == sc) & (iq_col < slot))
    g2bf = g2.astype(jnp.bfloat16)

    CH = 256
    for ib in range(BUF // CH):
        yb = jax.lax.slice(y_col, (ib * CH, 0), ((ib + 1) * CH, 1))
        xb = jax.lax.slice(x_col, (ib * CH, 0), ((ib + 1) * CH, 1))
        dy = yb - yc
        dx = xb - xc
        d2 = dy * dy + dx * dx
        g2b = jax.lax.slice(g2, (ib * CH, 0), ((ib + 1) * CH, BUF))
        S_s[ib * CH:(ib + 1) * CH, :] = (
            (d2 < THR2) & g2b).astype(jnp.bfloat16)

    keep_s[...] = jnp.ones((1, BUF), _f32)

    def ncond(st):
        changed, t = st
        return jnp.logical_and(changed, t < BUF + 4)

    def nbody(st):
        _, t = st
        k_cur = keep_s[...]
        sup = jax.lax.dot_general(
            k_cur.astype(jnp.bfloat16), S_s[...], (((1,), (0,)), ((), ())),
            preferred_element_type=_f32)
        k_new = (sup < 0.5).astype(_f32)
        keep_s[...] = k_new
        return (jnp.sum(jnp.abs(k_new - k_cur)) > 0.0, t + 1)

    jax.lax.while_loop(ncond, nbody, (True, jnp.int32(0)))
    keep = keep_s[...]

    keepreal = keep * real.astype(_f32)
    kcnt = jnp.sum(keepreal)
    outpos = jax.lax.dot_general(
        keepreal.astype(jnp.bfloat16), g2bf, (((1,), (0,)), ((), ())),
        preferred_element_type=_f32)
    oq_col = _fiota((OUTP, 1), 0)
    o_out = (oq_col == outpos).astype(_f32) * keepreal
    outv = _dotT(vc, o_out, exact=True)

    qi = _fiota((1, OUTP), 1)
    s_o = jnp.where(qi < kcnt, outv[0:1, :], -1.0)
    valid = s_o >= MIN_SCORE
    out_s_ref[0] = jnp.where(valid, s_o, -1.0)
    out_y_ref[0] = jnp.where(valid, outv[1:2, :], -1.0)
    out_x_ref[0] = jnp.where(valid, outv[2:3, :], -1.0)


def kernel(scores_out, regression_out, gt_locations):
    del gt_locations
    B = scores_out.shape[0]
    reg_y = regression_out[..., 0]
    reg_x = regression_out[..., 1]
    img_spec = pl.BlockSpec((1, H, W), lambda b: (b, 0, 0))
    out_spec = pl.BlockSpec((1, 1, OUTP), lambda b: (b, 0, 0))
    s, y, x = pl.pallas_call(
        _nms_body,
        grid=(B,),
        in_specs=[img_spec, img_spec, img_spec],
        out_specs=[out_spec, out_spec, out_spec],
        out_shape=[jax.ShapeDtypeStruct((B, 1, OUTP), _f32)] * 3,
        scratch_shapes=[pltpu.VMEM((H, W), _f32)] * 4 +
                       [pltpu.VMEM((BUF, BUF), jnp.bfloat16),
                        pltpu.VMEM((1, BUF), _f32),
                        pltpu.VMEM((3, BUF // W + 1, W), _f32)],
        compiler_params=pltpu.CompilerParams(
            vmem_limit_bytes=128 * 1024 * 1024),
    )(scores_out, reg_y, reg_x)
    locs = jnp.stack([y[:, 0, :OUTN], x[:, 0, :OUTN]], axis=-1)
    return locs, s[:, 0, :OUTN]


# speedup vs baseline: 1.8024x; 1.8024x over previous
import jax
import jax.numpy as jnp
from jax.experimental import pallas as pl
from jax.experimental.pallas import tpu as pltpu

H = 160
W = 128
N = H * W
KTOP = 2000
BUF = 2048
OUTN = 500
OUTP = 512
THR2 = 1.0
MIN_SCORE = 0.5
ONE_BITS = 0x3F800000

_f32 = jnp.float32


def _fiota(shape, dim):
    return jax.lax.broadcasted_iota(jnp.int32, shape, dim).astype(_f32)


def _dotT(a, b, exact=False):
    prec = jax.lax.Precision.HIGHEST if exact else jax.lax.Precision.DEFAULT
    return jax.lax.dot_general(
        a, b, (((1,), (1,)), ((), ())), precision=prec,
        preferred_element_type=_f32)


def _tril_strict(n):
    r = _fiota((n, n), 0)
    c = _fiota((n, n), 1)
    return (c < r).astype(_f32)


def _nms_body(scores_ref, reg_y_ref, reg_x_ref,
              out_s_ref, out_y_ref, out_x_ref,
              S_s, keep_s):
    scores = scores_ref[0]
    ri = _fiota((H, W), 0)
    ci = _fiota((H, W), 1)
    y2d = ri + 0.5 + reg_y_ref[0]
    x2d = ci + 0.5 + reg_x_ref[0]

    s_bits = jax.lax.bitcast_convert_type(scores, jnp.int32)

    def bis(_, lohi):
        lo, hi = lohi
        m = lo + (hi - lo) // 2
        cnt = jnp.sum((s_bits >= m).astype(jnp.int32))
        big = cnt >= KTOP
        return (jnp.where(big, m, lo), jnp.where(big, hi, m))

    lo, _ = jax.lax.fori_loop(
        0, 31, bis, (jnp.int32(0), jnp.int32(ONE_BITS)))
    mask_gt = (s_bits > lo).astype(_f32)
    mask_eq = (s_bits == lo).astype(_f32)
    need_eq = _f32(KTOP) - jnp.sum(mask_gt)

    TT_W = _tril_strict(W)
    TL_H = _tril_strict(H)

    def excl2d(m2d):
        e = _dotT(m2d, TT_W)
        t = jnp.sum(m2d, axis=1, keepdims=True)
        roff = jax.lax.dot_general(
            TL_H, t, (((1,), (0,)), ((), ())), preferred_element_type=_f32)
        return roff + e

    eq_pos = excl2d(mask_eq)
    mask_sel = mask_gt + mask_eq * (eq_pos < need_eq).astype(_f32)
    pos_e = _dotT(mask_sel, TT_W)

    bf16 = jnp.bfloat16
    t_sel = jnp.sum(mask_sel, axis=1, keepdims=True)
    roff_col = jax.lax.dot_general(
        TL_H, t_sel, (((1,), (0,)), ((), ())), preferred_element_type=_f32)
    both = jnp.concatenate([roff_col, roff_col + t_sel], axis=1)
    I_H = (_fiota((H, H), 0) == _fiota((H, H), 1)).astype(_f32)
    rows2 = jax.lax.dot_general(
        both, I_H, (((0,), (0,)), ((), ())),
        precision=jax.lax.Precision.HIGHEST, preferred_element_type=_f32)
    roff_row = rows2[0:1, :]
    rend_row = rows2[1:2, :]

    iq_col = _fiota((BUF, 1), 0)
    orf = ((roff_row <= iq_col) & (iq_col < rend_row)).astype(_f32)
    orbf = orf.astype(bf16)

    es = jnp.concatenate([pos_e, mask_sel], axis=1).astype(bf16)
    a_es = jax.lax.dot_general(
        orbf, es, (((1,), (0,)), ((), ())), preferred_element_type=_f32)
    a_e = a_es[:, 0:W]
    a_sel = a_es[:, W:2 * W]

    v3l = jnp.concatenate([scores, y2d, x2d], axis=1)
    v0 = v3l.astype(bf16)
    r1 = v3l - v0.astype(_f32)
    v1 = r1.astype(bf16)
    v2 = (r1 - v1.astype(_f32)).astype(bf16)
    a_v = sum(jax.lax.dot_general(
        orbf, vk, (((1,), (0,)), ((), ())), preferred_element_type=_f32)
        for vk in (v0, v1, v2))

    roff_q = jax.lax.dot_general(
        orf, roff_col, (((1,), (0,)), ((), ())),
        precision=jax.lax.Precision.HIGHEST, preferred_element_type=_f32)
    oc = ((a_sel > 0.5) & (a_e == (iq_col - roff_q))).astype(_f32)

    real_col = iq_col < _f32(KTOP)
    s_col = jnp.where(real_col,
                      jnp.sum(a_v[:, 0:W] * oc, axis=1, keepdims=True), -1.0)
    y_col = jnp.where(real_col,
                      jnp.sum(a_v[:, W:2 * W] * oc, axis=1, keepdims=True),
                      1e6 + iq_col)
    x_col = jnp.where(real_col,
                      jnp.sum(a_v[:, 2 * W:3 * W] * oc, axis=1, keepdims=True),
                      1e6)

    slot = _fiota((1, BUF), 1)
    real = slot < _f32(KTOP)
    I_BUF = (_fiota((BUF, BUF), 0) ==
             _fiota((BUF, BUF), 1)).astype(_f32)
    cols3 = jnp.concatenate([s_col, y_col, x_col], axis=1)
    vc = jax.lax.dot_general(
        cols3, I_BUF, (((0,), (0,)), ((), ())),
        precision=jax.lax.Precision.HIGHEST, preferred_element_type=_f32)
    sc = vc[0:1, :]
    yc = vc[1:2, :]
    xc = vc[2:3, :]

    g2 = (s_col > sc) | ((s_col == sc) & (iq_col < slot))
    g2bf = g2.astype(jnp.bfloat16)

    CH = 256
    for ib in range(BUF // CH):
        yb = jax.lax.slice(y_col, (ib * CH, 0), ((ib + 1) * CH, 1))
        xb = jax.lax.slice(x_col, (ib * CH, 0), ((ib + 1) * CH, 1))
        dy = yb - yc
        dx = xb - xc
        d2 = dy * dy + dx * dx
        g2b = jax.lax.slice(g2, (ib * CH, 0), ((ib + 1) * CH, BUF))
        S_s[ib * CH:(ib + 1) * CH, :] = (
            (d2 < THR2) & g2b).astype(jnp.bfloat16)

    keep_s[...] = jnp.ones((1, BUF), _f32)

    def ncond(st):
        changed, t = st
        return jnp.logical_and(changed, t < BUF + 4)

    def nbody(st):
        _, t = st
        k_cur = keep_s[...]
        sup = jax.lax.dot_general(
            k_cur.astype(jnp.bfloat16), S_s[...], (((1,), (0,)), ((), ())),
            preferred_element_type=_f32)
        k_new = (sup < 0.5).astype(_f32)
        keep_s[...] = k_new
        return (jnp.sum(jnp.abs(k_new - k_cur)) > 0.0, t + 1)

    jax.lax.while_loop(ncond, nbody, (True, jnp.int32(0)))
    keep = keep_s[...]

    keepreal = keep * real.astype(_f32)
    kcnt = jnp.sum(keepreal)
    outpos = jax.lax.dot_general(
        keepreal.astype(jnp.bfloat16), g2bf, (((1,), (0,)), ((), ())),
        preferred_element_type=_f32)
    oq_col = _fiota((OUTP, 1), 0)
    o_out = (oq_col == outpos).astype(_f32) * keepreal
    outv = _dotT(vc, o_out, exact=True)

    qi = _fiota((1, OUTP), 1)
    s_o = jnp.where(qi < kcnt, outv[0:1, :], -1.0)
    valid = s_o >= MIN_SCORE
    out_s_ref[0] = jnp.where(valid, s_o, -1.0)
    out_y_ref[0] = jnp.where(valid, outv[1:2, :], -1.0)
    out_x_ref[0] = jnp.where(valid, outv[2:3, :], -1.0)


def kernel(scores_out, regression_out, gt_locations):
    del gt_locations
    B = scores_out.shape[0]
    reg_y = regression_out[..., 0]
    reg_x = regression_out[..., 1]
    img_spec = pl.BlockSpec((1, H, W), lambda b: (b, 0, 0))
    out_spec = pl.BlockSpec((1, 1, OUTP), lambda b: (b, 0, 0))
    s, y, x = pl.pallas_call(
        _nms_body,
        grid=(B,),
        in_specs=[img_spec, img_spec, img_spec],
        out_specs=[out_spec, out_spec, out_spec],
        out_shape=[jax.ShapeDtypeStruct((B, 1, OUTP), _f32)] * 3,
        scratch_shapes=[pltpu.VMEM((BUF, BUF), jnp.bfloat16),
                        pltpu.VMEM((1, BUF), _f32)],
        compiler_params=pltpu.CompilerParams(
            vmem_limit_bytes=128 * 1024 * 1024),
    )(scores_out, reg_y, reg_x)
    locs = jnp.stack([y[:, 0, :OUTN], x[:, 0, :OUTN]], axis=-1)
    return locs, s[:, 0, :OUTN]


# speedup vs baseline: 1.8028x; 1.0002x over previous
import jax
import jax.numpy as jnp
from jax.experimental import pallas as pl
from jax.experimental.pallas import tpu as pltpu

H = 160
W = 128
N = H * W
KTOP = 2000
BUF = 2048
OUTN = 500
OUTP = 512
THR2 = 1.0
MIN_SCORE = 0.5
ONE_BITS = 0x3F800000

_f32 = jnp.float32


def _fiota(shape, dim):
    return jax.lax.broadcasted_iota(jnp.int32, shape, dim).astype(_f32)


def _dotT(a, b, exact=False):
    prec = jax.lax.Precision.HIGHEST if exact else jax.lax.Precision.DEFAULT
    return jax.lax.dot_general(
        a, b, (((1,), (1,)), ((), ())), precision=prec,
        preferred_element_type=_f32)


def _tril_strict(n):
    r = _fiota((n, n), 0)
    c = _fiota((n, n), 1)
    return (c < r).astype(_f32)


def _nms_body(scores_ref, reg_y_ref, reg_x_ref,
              out_s_ref, out_y_ref, out_x_ref,
              S_s, keep_s):
    scores = scores_ref[0]
    ri = _fiota((H, W), 0)
    ci = _fiota((H, W), 1)
    y2d = ri + 0.5 + reg_y_ref[0]
    x2d = ci + 0.5 + reg_x_ref[0]

    s_bits = jax.lax.bitcast_convert_type(scores, jnp.int32)

    def bis(_, lohi):
        lo, hi = lohi
        m = lo + (hi - lo) // 2
        cnt = jnp.sum((s_bits >= m).astype(jnp.int32))
        big = cnt >= KTOP
        return (jnp.where(big, m, lo), jnp.where(big, hi, m))

    lo, _ = jax.lax.fori_loop(
        0, 31, bis, (jnp.int32(0), jnp.int32(ONE_BITS)))
    mask_gt = (s_bits > lo).astype(_f32)
    mask_eq = (s_bits == lo).astype(_f32)
    need_eq = _f32(KTOP) - jnp.sum(mask_gt)

    TT_W = _tril_strict(W)
    TL_H = _tril_strict(H)

    def excl2d(m2d):
        e = _dotT(m2d, TT_W)
        t = jnp.sum(m2d, axis=1, keepdims=True)
        roff = jax.lax.dot_general(
            TL_H, t, (((1,), (0,)), ((), ())), preferred_element_type=_f32)
        return roff + e

    eq_pos = excl2d(mask_eq)
    mask_sel = mask_gt + mask_eq * (eq_pos < need_eq).astype(_f32)
    pos_e = _dotT(mask_sel, TT_W)

    bf16 = jnp.bfloat16
    t_sel = jnp.sum(mask_sel, axis=1, keepdims=True)
    roff_col = jax.lax.dot_general(
        TL_H, t_sel, (((1,), (0,)), ((), ())), preferred_element_type=_f32)
    both = jnp.concatenate([roff_col, roff_col + t_sel], axis=1)
    I_H = (_fiota((H, H), 0) == _fiota((H, H), 1)).astype(_f32)
    rows2 = jax.lax.dot_general(
        both, I_H, (((0,), (0,)), ((), ())),
        precision=jax.lax.Precision.HIGHEST, preferred_element_type=_f32)
    roff_row = rows2[0:1, :]
    rend_row = rows2[1:2, :]

    iq_col = _fiota((BUF, 1), 0)
    orf = ((roff_row <= iq_col) & (iq_col < rend_row)).astype(_f32)
    orbf = orf.astype(bf16)

    es = jnp.concatenate([pos_e, mask_sel], axis=1).astype(bf16)
    a_es = jax.lax.dot_general(
        orbf, es, (((1,), (0,)), ((), ())), preferred_element_type=_f32)
    a_e = a_es[:, 0:W]
    a_sel = a_es[:, W:2 * W]

    v3l = jnp.concatenate([scores, y2d, x2d], axis=1)
    v0 = v3l.astype(bf16)
    r1 = v3l - v0.astype(_f32)
    v1 = r1.astype(bf16)
    v2 = (r1 - v1.astype(_f32)).astype(bf16)
    a_v = sum(jax.lax.dot_general(
        orbf, vk, (((1,), (0,)), ((), ())), preferred_element_type=_f32)
        for vk in (v0, v1, v2))

    roff_q = jax.lax.dot_general(
        orf, roff_col, (((1,), (0,)), ((), ())),
        precision=jax.lax.Precision.HIGHEST, preferred_element_type=_f32)
    oc = ((a_sel > 0.5) & (a_e == (iq_col - roff_q))).astype(_f32)

    real_col = iq_col < _f32(KTOP)
    s_col = jnp.where(real_col,
                      jnp.sum(a_v[:, 0:W] * oc, axis=1, keepdims=True), -1.0)
    y_col = jnp.where(real_col,
                      jnp.sum(a_v[:, W:2 * W] * oc, axis=1, keepdims=True),
                      1e6 + iq_col)
    x_col = jnp.where(real_col,
                      jnp.sum(a_v[:, 2 * W:3 * W] * oc, axis=1, keepdims=True),
                      1e6)

    slot = _fiota((1, BUF), 1)
    real = slot < _f32(KTOP)
    I_BUF = (_fiota((BUF, BUF), 0) ==
             _fiota((BUF, BUF), 1)).astype(_f32)
    cols3 = jnp.concatenate([s_col, y_col, x_col], axis=1)
    vc = jax.lax.dot_general(
        cols3, I_BUF, (((0,), (0,)), ((), ())),
        precision=jax.lax.Precision.HIGHEST, preferred_element_type=_f32)
    sc = vc[0:1, :]
    yc = vc[1:2, :]
    xc = vc[2:3, :]

    g2 = (s_col > sc) | ((s_col == sc) & (iq_col < slot))
    g2bf = g2.astype(jnp.bfloat16)

    CH = 256
    for ib in range(BUF // CH):
        yb = jax.lax.slice(y_col, (ib * CH, 0), ((ib + 1) * CH, 1))
        xb = jax.lax.slice(x_col, (ib * CH, 0), ((ib + 1) * CH, 1))
        dy = yb - yc
        dx = xb - xc
        d2 = dy * dy + dx * dx
        g2b = jax.lax.slice(g2, (ib * CH, 0), ((ib + 1) * CH, BUF))
        S_s[ib * CH:(ib + 1) * CH, :] = (
            (d2 < THR2) & g2b).astype(jnp.bfloat16)

    keep_s[...] = jnp.ones((1, BUF), _f32)

    def ncond(st):
        changed, t = st
        return jnp.logical_and(changed, t < BUF + 4)

    def nbody(st):
        _, t = st
        k_cur = keep_s[...]
        sup = jax.lax.dot_general(
            k_cur.astype(jnp.bfloat16), S_s[...], (((1,), (0,)), ((), ())),
            preferred_element_type=_f32)
        k_new = (sup < 0.5).astype(_f32)
        keep_s[...] = k_new
        return (jnp.sum(jnp.abs(k_new - k_cur)) > 0.0, t + 1)

    jax.lax.while_loop(ncond, nbody, (True, jnp.int32(0)))
    keep = keep_s[...]

    keepreal = keep * real.astype(_f32)
    kcnt = jnp.sum(keepreal)
    outpos = jax.lax.dot_general(
        keepreal.astype(jnp.bfloat16), g2bf, (((1,), (0,)), ((), ())),
        preferred_element_type=_f32)
    oq_col = _fiota((OUTP, 1), 0)
    o_out = (oq_col == outpos).astype(_f32) * keepreal
    outv = _dotT(vc, o_out, exact=True)

    qi = _fiota((1, OUTP), 1)
    s_o = jnp.where(qi < kcnt, outv[0:1, :], -1.0)
    valid = s_o >= MIN_SCORE
    out_s_ref[0] = jnp.where(valid, s_o, -1.0)
    out_y_ref[0] = jnp.where(valid, outv[1:2, :], -1.0)
    out_x_ref[0] = jnp.where(valid, outv[2:3, :], -1.0)


def kernel(scores_out, regression_out, gt_locations):
    del gt_locations
    B = scores_out.shape[0]
    reg_y = regression_out[..., 0]
    reg_x = regression_out[..., 1]
    img_spec = pl.BlockSpec((1, H, W), lambda b: (b, 0, 0))
    out_spec = pl.BlockSpec((1, 1, OUTP), lambda b: (b, 0, 0))
    s, y, x = pl.pallas_call(
        _nms_body,
        grid=(B,),
        in_specs=[img_spec, img_spec, img_spec],
        out_specs=[out_spec, out_spec, out_spec],
        out_shape=[jax.ShapeDtypeStruct((B, 1, OUTP), _f32)] * 3,
        scratch_shapes=[pltpu.VMEM((BUF, BUF), jnp.bfloat16),
                        pltpu.VMEM((1, BUF), _f32)],
        compiler_params=pltpu.CompilerParams(
            dimension_semantics=("parallel",),
            vmem_limit_bytes=128 * 1024 * 1024),
    )(scores_out, reg_y, reg_x)
    locs = jnp.stack([y[:, 0, :OUTN], x[:, 0, :OUTN]], axis=-1)
    return locs, s[:, 0, :OUTN]
